# Initial kernel scaffold; baseline (speedup 1.0000x reference)
import jax
import jax.numpy as jnp
from jax.experimental import pallas as pl


def kernel(features, edge_index, W1_src, W1_dst, att1_src, att1_dst, W2_src, W2_dst, att2_src, att2_dst, att3_src, att3_dst, smooth_alpha_1, smooth_alpha_2):
    raise NotImplementedError("write your pallas kernel here")


# speedup vs baseline: 1.3254x; 1.3254x over previous
import functools

import jax
import jax.numpy as jnp
from jax.experimental import pallas as pl
from jax.experimental.pallas import tpu as pltpu

N = 10000
NP = 10240


def _combine_body(a_ref, o_ref, *, k, scale):
    acc = a_ref[0]
    for i in range(1, k):
        acc = acc + a_ref[i]
    o_ref[...] = acc * scale


def _combine(parts, scale=1.0):
    k, np_, w = parts.shape
    blk = 1024
    return pl.pallas_call(
        functools.partial(_combine_body, k=k, scale=scale),
        out_shape=jax.ShapeDtypeStruct((np_, w), jnp.float32),
        grid=(np_ // blk,),
        in_specs=[pl.BlockSpec((k, blk, w), lambda i: (0, i, 0))],
        out_specs=pl.BlockSpec((blk, w), lambda i: (i, 0)),
    )(parts)


def kernel(features, edge_index, W1_src, W1_dst, att1_src, att1_dst,
           W2_src, W2_dst, att2_src, att2_dst, att3_src, att3_dst,
           smooth_alpha_1, smooth_alpha_2):
    n = features.shape[0]
    src, dst = edge_index[0], edge_index[1]

    def seg_softmax(a_s, a_d):
        e = jax.nn.leaky_relu(a_s[src] + a_d[dst], negative_slope=0.2)
        e_exp = jnp.exp(e)
        denom = jax.ops.segment_sum(e_exp, dst, num_segments=n)
        return e_exp / (denom[dst] + 1e-16)

    def spmm(tbl, w):
        return jax.ops.segment_sum(w[:, None] * tbl[src], dst, num_segments=n)

    x = features
    a1s = x @ (W1_src @ att1_src); a1d = x @ (W1_dst @ att1_dst)
    al1 = seg_softmax(a1s, a1d)
    m1 = spmm(x, al1)
    h1 = jax.nn.selu(m1 @ W1_src)
    a2s = h1 @ (W2_src @ att2_src); a2d = h1 @ (W2_dst @ att2_dst)
    xs2 = h1 @ W2_src
    al2 = seg_softmax(a2s, a2d)
    h2 = spmm(xs2, al2)
    a3s = h2 @ (W2_src.T @ att3_src); a3d = h2 @ (W2_dst.T @ att3_dst)
    al3 = seg_softmax(a3s, a3d)
    m3 = spmm(h2, al3)
    h3 = jax.nn.selu(m3 @ W2_src.T)
    h4 = h3 @ W1_src.T

    row, col = src, dst
    deg = jax.ops.segment_sum(jnp.ones_like(al1), row, num_segments=n)
    dis = jnp.where(deg > 0, deg ** (-0.6), 0.0)
    base = dis[row] * dis[col]

    def smooth(tbl, w):
        return jax.ops.segment_sum(w[:, None] * tbl[col], row, num_segments=n)

    f1 = f2 = h2
    w1 = base * smooth_alpha_1; w2 = base * smooth_alpha_2 * 2.0
    for _ in range(2):
        f1 = smooth(f1, w1); f2 = smooth(f2, w2)
    dis2 = jnp.where(deg > 0, deg ** (-0.5), 0.0)
    alpha_d = 1.0 / (1.0 + jnp.log(deg + 1.0))
    wn = dis2[row] * dis2[col] * alpha_d[row]

    def pad(a):
        return jnp.pad(a, ((0, NP - n), (0, 32 - a.shape[1])))
    sm = _combine(jnp.stack([pad(f1), pad(f2)]), 0.5)[:n, :30]
    for _ in range(2):
        sm = smooth(sm, wn)
    return (sm, h4)


# speedup vs baseline: 22.3763x; 16.8832x over previous
import functools

import jax
import jax.numpy as jnp
from jax import lax
from jax.experimental import pallas as pl
from jax.experimental.pallas import tpu as pltpu
from jax.experimental.pallas import tpu_sc as plsc

N = 10000
E = 320000
NP = 10240
B = 128
NB = 79
EPT = NB * B
EPAD = 32 * EPT
SLICE = NP // 16

_SELU_SCALE = 1.0507009873554805
_SELU_ALPHA = 1.6732632423543772


def _selu(x):
    return _SELU_SCALE * jnp.where(x > 0, x, _SELU_ALPHA * (jnp.exp(x) - 1.0))


def _zero_rows(rows, nrow, w):
    def zr(i, _):
        def zc(c, _):
            rows[i, pl.ds(c * 16, 16)] = jnp.zeros((16,), jnp.float32)
            return 0
        return lax.fori_loop(0, w // 16, zc, 0)
    lax.fori_loop(0, nrow, zr, 0)


def _zero_vec(buf, n):
    def zb(i, _):
        buf[pl.ds(i * 16, 16)] = jnp.zeros((16,), jnp.float32)
        return 0
    lax.fori_loop(0, n // 16, zb, 0)


def _gat_body(W, splits, with_deg, srcp, dstp, a_s_h, a_d_h, *rest):
    Wc = W // splits
    tables = rest[:splits]
    rest = rest[splits:]
    out_h = rest[0]
    rest = rest[1:]
    if with_deg:
        deg_h = rest[0]
        rest = rest[1:]
    (a_s, a_d, src_l, dst_l, ee_l, dn_l, rows, zbuf, ones_b,
     acc, dn_acc) = rest[:11]
    rest = rest[11:]
    if with_deg:
        deg_acc = rest[0]
        rest = rest[1:]
    sem = rest[0]

    cid = lax.axis_index("c")
    sid = lax.axis_index("s")

    _zero_vec(zbuf, SLICE)
    pltpu.sync_copy(zbuf, dn_acc.at[pl.ds(sid * SLICE, SLICE)])
    if with_deg:
        pltpu.sync_copy(zbuf, deg_acc.at[pl.ds(sid * SLICE, SLICE)])

    pltpu.sync_copy(a_s_h, a_s)
    pltpu.sync_copy(a_d_h, a_d)
    def one16(i, _):
        ones_b[pl.ds(i * 16, 16)] = jnp.ones((16,), jnp.float32)
        return 0
    lax.fori_loop(0, B // 16, one16, 0)

    plsc.subcore_barrier()

    for t in range(2):
        chunk = sid * 2 + jnp.where(jnp.int32(t) == 0, 1 - cid, cid)
        pltpu.sync_copy(srcp.at[chunk], src_l)
        pltpu.sync_copy(dstp.at[chunk], dst_l)

        def logits(j, _):
            def inner(k, _):
                sl = pl.ds(k * 16, 16)
                sv = src_l[j, sl]
                dv = dst_l[j, sl]
                av = plsc.load_gather(a_s, [sv])
                bv = plsc.load_gather(a_d, [dv])
                e = av + bv
                e = jnp.where(e > 0, e, 0.2 * e)
                ee_l[j, sl] = jnp.exp(e)
                return 0
            return lax.fori_loop(0, B // 16, inner, 0)
        lax.fori_loop(0, NB, logits, 0)

        descs = []
        for j in range(NB):
            descs.append(pltpu.async_copy(
                ee_l.at[j], dn_acc.at[dst_l.at[j]], sem, add=True))
            if with_deg:
                descs.append(pltpu.async_copy(
                    ones_b, deg_acc.at[src_l.at[j]], sem, add=True))
        for d in descs:
            d.wait()

    plsc.subcore_barrier()

    pltpu.sync_copy(dn_acc, dn_l)

    def alphas(j, _):
        def inner(k, _):
            sl = pl.ds(k * 16, 16)
            dv = dst_l[j, sl]
            dn = plsc.load_gather(dn_l, [dv])
            ee_l[j, sl] = ee_l[j, sl] / (dn + 1e-16)
            return 0
        return lax.fori_loop(0, B // 16, inner, 0)
    lax.fori_loop(0, NB, alphas, 0)

    for half, table_h in enumerate(tables):
        _zero_rows(rows, B, Wc)
        for q in range(SLICE // B):
            pltpu.sync_copy(rows, acc.at[pl.ds(sid * SLICE + q * B, B)])
        plsc.subcore_barrier()

        def spmm(j, _):
            pltpu.async_copy(table_h.at[src_l.at[j]], rows, sem).wait()

            def scale(r, _):
                a = plsc.load_gather(ee_l, [jnp.full((16,), j, jnp.int32),
                                            jnp.full((16,), r, jnp.int32)])
                for c in range(Wc // 16):
                    sl = pl.ds(c * 16, 16)
                    rows[r, sl] = rows[r, sl] * a
                return 0
            lax.fori_loop(0, B, scale, 0)
            pltpu.sync_copy(rows, acc.at[dst_l.at[j]], add=True)
            return 0
        lax.fori_loop(0, NB, spmm, 0)

        plsc.subcore_barrier()

        sl = pl.ds(sid * SLICE, SLICE)
        if splits == 1:
            pltpu.sync_copy(acc.at[sl], out_h.at[cid, sl])
        else:
            pltpu.sync_copy(acc.at[sl], out_h.at[cid, half, sl])

    if with_deg:
        sl = pl.ds(sid * SLICE, SLICE)
        @pl.when(cid == 0)
        def _():
            pltpu.sync_copy(deg_acc.at[sl], deg_h.at[sl])


def _sc_gat(W, with_deg):
    splits = 2 if W > 64 else 1
    Wc = W // splits
    mesh = plsc.VectorSubcoreMesh(core_axis_name="c", subcore_axis_name="s")
    out_shape = (2, NP, W) if splits == 1 else (2, splits, NP, Wc)
    out_type = [jax.ShapeDtypeStruct(out_shape, jnp.float32)]
    if with_deg:
        out_type.append(jax.ShapeDtypeStruct((NP,), jnp.float32))
    scratch = [
        pltpu.VMEM((NP,), jnp.float32),
        pltpu.VMEM((NP,), jnp.float32),
        pltpu.VMEM((NB, B), jnp.int32),
        pltpu.VMEM((NB, B), jnp.int32),
        pltpu.VMEM((NB, B), jnp.float32),
        pltpu.VMEM((NP,), jnp.float32),
        pltpu.VMEM((B, Wc), jnp.float32),
        pltpu.VMEM((SLICE,), jnp.float32),
        pltpu.VMEM((B,), jnp.float32),
        pltpu.VMEM_SHARED((NP, Wc), jnp.float32),
        pltpu.VMEM_SHARED((NP,), jnp.float32),
    ]
    if with_deg:
        scratch.append(pltpu.VMEM_SHARED((NP,), jnp.float32))
    scratch.append(pltpu.SemaphoreType.DMA)
    return pl.kernel(
        functools.partial(_gat_body, W, splits, with_deg),
        out_type=tuple(out_type) if len(out_type) > 1 else out_type[0],
        mesh=mesh,
        scratch_types=tuple(scratch),
        compiler_params=pltpu.CompilerParams(needs_layout_passes=False, use_tc_tiling_on_sc=False),
    )


def _smooth_body(n_out, share, colp, rowp, ta_h, tb_h, *args):
    if n_out == 2:
        ta2_h = args[0]
        args = args[1:]
    t1_h = args[0]
    args = args[1:]
    if n_out == 2 and not share:
        t2_h = args[0]
        args = args[1:]
    out1_h = args[0]
    args = args[1:]
    if n_out == 2:
        out2_h = args[0]
        args = args[1:]
    (ta, tb, col_l, row_l, w1_l, rows1) = args[:6]
    args = args[6:]
    if n_out == 2:
        ta2, w2_l, rows2, acc2 = args[:4]
        args = args[4:]
    acc1 = args[0]
    sem = args[1]

    cid = lax.axis_index("c")
    sid = lax.axis_index("s")
    wid = sid * 2 + cid

    Wd = 32
    _zero_rows(rows1, B, Wd)
    for q in range(SLICE // B):
        sl = pl.ds(sid * SLICE + q * B, B)
        pltpu.sync_copy(rows1, acc1.at[sl])
        if n_out == 2:
            pltpu.sync_copy(rows1, acc2.at[sl])

    pltpu.sync_copy(ta_h, ta)
    pltpu.sync_copy(tb_h, tb)
    if n_out == 2:
        pltpu.sync_copy(ta2_h, ta2)
    pltpu.sync_copy(colp.at[wid], col_l)
    pltpu.sync_copy(rowp.at[wid], row_l)

    def weights(j, _):
        def inner(k, _):
            sl = pl.ds(k * 16, 16)
            rv = row_l[j, sl]
            cv = col_l[j, sl]
            tbv = plsc.load_gather(tb, [cv])
            w1_l[j, sl] = plsc.load_gather(ta, [rv]) * tbv
            if n_out == 2:
                w2_l[j, sl] = plsc.load_gather(ta2, [rv]) * tbv
            return 0
        return lax.fori_loop(0, B // 16, inner, 0)
    lax.fori_loop(0, NB, weights, 0)

    plsc.subcore_barrier()

    def spmm(j, _):
        pltpu.async_copy(t1_h.at[col_l.at[j]], rows1, sem).wait()
        if n_out == 2 and not share:
            pltpu.async_copy(t2_h.at[col_l.at[j]], rows2, sem).wait()

        def scale(r, _):
            jr = [jnp.full((16,), j, jnp.int32), jnp.full((16,), r, jnp.int32)]
            a1 = plsc.load_gather(w1_l, jr)
            if n_out == 2:
                a2 = plsc.load_gather(w2_l, jr)
            for c in range(Wd // 16):
                sl = pl.ds(c * 16, 16)
                v1 = rows1[r, sl]
                if n_out == 2:
                    v2 = v1 if share else rows2[r, sl]
                    rows2[r, sl] = v2 * a2
                rows1[r, sl] = v1 * a1
            return 0
        lax.fori_loop(0, B, scale, 0)
        pltpu.sync_copy(rows1, acc1.at[row_l.at[j]], add=True)
        if n_out == 2:
            pltpu.sync_copy(rows2, acc2.at[row_l.at[j]], add=True)
        return 0
    lax.fori_loop(0, NB, spmm, 0)

    plsc.subcore_barrier()

    sl = pl.ds(sid * SLICE, SLICE)
    pltpu.sync_copy(acc1.at[sl], out1_h.at[cid, sl])
    if n_out == 2:
        pltpu.sync_copy(acc2.at[sl], out2_h.at[cid, sl])


def _sc_smooth(n_out, share):
    mesh = plsc.VectorSubcoreMesh(core_axis_name="c", subcore_axis_name="s")
    Wd = 32
    outs = [jax.ShapeDtypeStruct((2, NP, Wd), jnp.float32)] * n_out
    scratch = [
        pltpu.VMEM((NP,), jnp.float32),
        pltpu.VMEM((NP,), jnp.float32),
        pltpu.VMEM((NB, B), jnp.int32),
        pltpu.VMEM((NB, B), jnp.int32),
        pltpu.VMEM((NB, B), jnp.float32),
        pltpu.VMEM((B, Wd), jnp.float32),
    ]
    if n_out == 2:
        scratch += [
            pltpu.VMEM((NP,), jnp.float32),
            pltpu.VMEM((NB, B), jnp.float32),
            pltpu.VMEM((B, Wd), jnp.float32),
            pltpu.VMEM_SHARED((NP, Wd), jnp.float32),
        ]
    scratch += [
        pltpu.VMEM_SHARED((NP, Wd), jnp.float32),
        pltpu.SemaphoreType.DMA,
    ]
    return pl.kernel(
        functools.partial(_smooth_body, n_out, share),
        out_type=tuple(outs) if n_out == 2 else outs[0],
        mesh=mesh,
        scratch_types=tuple(scratch),
        compiler_params=pltpu.CompilerParams(needs_layout_passes=False, use_tc_tiling_on_sc=False),
    )


_BLK = 1024
_G = NP // _BLK


def _tc0_body(x_ref, v1_ref, o1_ref, o2_ref):
    x = x_ref[...]
    o1_ref[...] = jnp.sum(x * v1_ref[0][None, :], axis=1)
    o2_ref[...] = jnp.sum(x * v1_ref[1][None, :], axis=1)


def _tc0(x_pad, v1s, v1d):
    return pl.pallas_call(
        _tc0_body,
        out_shape=(jax.ShapeDtypeStruct((NP,), jnp.float32),
                   jax.ShapeDtypeStruct((NP,), jnp.float32)),
        grid=(_G,),
        in_specs=[pl.BlockSpec((_BLK, 128), lambda i: (i, 0)),
                  pl.BlockSpec((2, 128), lambda i: (0, 0))],
        out_specs=(pl.BlockSpec((_BLK,), lambda i: (i,)),
                   pl.BlockSpec((_BLK,), lambda i: (i,))),
    )(x_pad, jnp.stack([v1s, v1d]))


def _tc1_body(p_ref, w1_ref, w2p_ref, v2_ref, xs2_ref, a2s_ref, a2d_ref):
    m = jnp.concatenate([p_ref[0, 0] + p_ref[1, 0],
                         p_ref[0, 1] + p_ref[1, 1]], axis=1)
    h1 = _selu(jnp.dot(m, w1_ref[...], preferred_element_type=jnp.float32))
    xs2_ref[...] = jnp.dot(h1, w2p_ref[...], preferred_element_type=jnp.float32)
    a2s_ref[...] = jnp.sum(h1 * v2_ref[0][None, :], axis=1)
    a2d_ref[...] = jnp.sum(h1 * v2_ref[1][None, :], axis=1)


def _tc1(m1P, W1_src, W2p, v2s, v2d):
    return pl.pallas_call(
        _tc1_body,
        out_shape=(jax.ShapeDtypeStruct((NP, 32), jnp.float32),
                   jax.ShapeDtypeStruct((NP,), jnp.float32),
                   jax.ShapeDtypeStruct((NP,), jnp.float32)),
        grid=(_G,),
        in_specs=[pl.BlockSpec((2, 2, _BLK, 64), lambda i: (0, 0, i, 0)),
                  pl.BlockSpec((128, 512), lambda i: (0, 0)),
                  pl.BlockSpec((512, 32), lambda i: (0, 0)),
                  pl.BlockSpec((2, 512), lambda i: (0, 0))],
        out_specs=(pl.BlockSpec((_BLK, 32), lambda i: (i, 0)),
                   pl.BlockSpec((_BLK,), lambda i: (i,)),
                   pl.BlockSpec((_BLK,), lambda i: (i,))),
    )(m1P, W1_src, W2p, jnp.stack([v2s, v2d]))


def _tc2_body(p_ref, deg_ref, v3_ref, h2_ref, a3s_ref, a3d_ref,
              dis_ref, dis2ad_ref, dis2_ref):
    h2 = p_ref[0] + p_ref[1]
    h2_ref[...] = h2
    a3s_ref[...] = jnp.sum(h2 * v3_ref[0][None, :], axis=1)
    a3d_ref[...] = jnp.sum(h2 * v3_ref[1][None, :], axis=1)
    deg = deg_ref[...]
    logd = jnp.log(jnp.maximum(deg, 1e-30))
    dis_ref[...] = jnp.where(deg > 0, jnp.exp(-0.6 * logd), 0.0)
    dis2 = jnp.where(deg > 0, jnp.exp(-0.5 * logd), 0.0)
    dis2_ref[...] = dis2
    alpha_d = 1.0 / (1.0 + jnp.log(deg + 1.0))
    dis2ad_ref[...] = dis2 * alpha_d


def _tc2(h2P, deg, v3s, v3d):
    return pl.pallas_call(
        _tc2_body,
        out_shape=(jax.ShapeDtypeStruct((NP, 32), jnp.float32),
                   jax.ShapeDtypeStruct((NP,), jnp.float32),
                   jax.ShapeDtypeStruct((NP,), jnp.float32),
                   jax.ShapeDtypeStruct((NP,), jnp.float32),
                   jax.ShapeDtypeStruct((NP,), jnp.float32),
                   jax.ShapeDtypeStruct((NP,), jnp.float32)),
        grid=(_G,),
        in_specs=[pl.BlockSpec((2, _BLK, 32), lambda i: (0, i, 0)),
                  pl.BlockSpec((_BLK,), lambda i: (i,)),
                  pl.BlockSpec((2, 32), lambda i: (0, 0))],
        out_specs=(pl.BlockSpec((_BLK, 32), lambda i: (i, 0)),
                   pl.BlockSpec((_BLK,), lambda i: (i,)),
                   pl.BlockSpec((_BLK,), lambda i: (i,)),
                   pl.BlockSpec((_BLK,), lambda i: (i,)),
                   pl.BlockSpec((_BLK,), lambda i: (i,)),
                   pl.BlockSpec((_BLK,), lambda i: (i,))),
    )(h2P, deg, jnp.stack([v3s, v3d]))


def _tc3_body(p_ref, w2t_ref, w1t_ref, h4_ref):
    m = p_ref[0] + p_ref[1]
    h3 = _selu(jnp.dot(m, w2t_ref[...], preferred_element_type=jnp.float32))
    h4_ref[...] = jnp.dot(h3, w1t_ref[...], preferred_element_type=jnp.float32)


def _tc3(m3P, W2pT, W1T):
    return pl.pallas_call(
        _tc3_body,
        out_shape=jax.ShapeDtypeStruct((NP, 128), jnp.float32),
        grid=(_G,),
        in_specs=[pl.BlockSpec((2, _BLK, 32), lambda i: (0, i, 0)),
                  pl.BlockSpec((32, 512), lambda i: (0, 0)),
                  pl.BlockSpec((512, 128), lambda i: (0, 0))],
        out_specs=pl.BlockSpec((_BLK, 128), lambda i: (i, 0)),
    )(m3P, W2pT, W1T)


def _combine_body(a_ref, o_ref, *, k, scale):
    acc = a_ref[0]
    for i in range(1, k):
        acc = acc + a_ref[i]
    o_ref[...] = acc * scale


def _combine(parts, scale=1.0):
    k, np_, w = parts.shape
    return pl.pallas_call(
        functools.partial(_combine_body, k=k, scale=scale),
        out_shape=jax.ShapeDtypeStruct((np_, w), jnp.float32),
        grid=(np_ // _BLK,),
        in_specs=[pl.BlockSpec((k, _BLK, w), lambda i: (0, i, 0))],
        out_specs=pl.BlockSpec((_BLK, w), lambda i: (i, 0)),
    )(parts)


def kernel(features, edge_index, W1_src, W1_dst, att1_src, att1_dst,
           W2_src, W2_dst, att2_src, att2_dst, att3_src, att3_dst,
           smooth_alpha_1, smooth_alpha_2):
    f32 = jnp.float32
    src = edge_index[0]
    dst = edge_index[1]
    pad_e = EPAD - E
    srcp = jnp.concatenate([src, jnp.full((pad_e,), N, jnp.int32)]
                           ).reshape(32, NB, B)
    dstp = jnp.concatenate([dst, jnp.full((pad_e,), N, jnp.int32)]
                           ).reshape(32, NB, B)

    x_pad = jnp.pad(features, ((0, NP - N), (0, 0)))
    W2p = jnp.pad(W2_src, ((0, 0), (0, 2)))
    W2pT = jnp.pad(W2_src.T, ((0, 2), (0, 0)))
    W1T = W1_src.T
    v1s = W1_src @ att1_src
    v1d = W1_dst @ att1_dst
    v2s = W2_src @ att2_src
    v2d = W2_dst @ att2_dst
    v3s = jnp.pad(W2_src.T @ att3_src, (0, 2))
    v3d = jnp.pad(W2_dst.T @ att3_dst, (0, 2))

    a1s, a1d = _tc0(x_pad, v1s, v1d)
    m1P, deg = _sc_gat(128, True)(srcp, dstp, a1s, a1d,
                                  x_pad[:, :64], x_pad[:, 64:])
    xs2, a2s, a2d = _tc1(m1P, W1_src, W2p, v2s, v2d)

    h2P = _sc_gat(32, False)(srcp, dstp, a2s, a2d, xs2)
    h2, a3s, a3d, dis, dis2ad, dis2 = _tc2(h2P, deg, v3s, v3d)

    m3P = _sc_gat(32, False)(srcp, dstp, a3s, a3d, h2)
    h4 = _tc3(m3P, W2pT, W1T)

    dis_a1 = dis * smooth_alpha_1
    dis_a2 = dis * (2.0 * smooth_alpha_2)
    f1P, f2P = _sc_smooth(2, True)(dstp, srcp, dis_a1, dis, dis_a2, h2)
    f1 = _combine(f1P)
    f2 = _combine(f2P)
    f1P, f2P = _sc_smooth(2, False)(dstp, srcp, dis_a1, dis, dis_a2, f1, f2)
    sm = _combine(jnp.concatenate([f1P, f2P]), 0.5)
    smP = _sc_smooth(1, False)(dstp, srcp, dis2ad, dis2, sm)
    sm = _combine(smP)
    smP = _sc_smooth(1, False)(dstp, srcp, dis2ad, dis2, sm)
    sm = _combine(smP)

    return (sm[:N, :30], h4[:N])
